# baseline (speedup 1.0000x reference)
import functools

import jax
import jax.numpy as jnp
from jax import lax
from jax.experimental import pallas as pl
from jax.experimental.pallas import tpu as pltpu
from jax.experimental.pallas import tpu_sc as plsc

BATCH = 4096
SEQ = 200
DIM = 64
B_TOTAL = BATCH * SEQ
NUM_WORKERS = 32
B_PER_W = B_TOTAL // NUM_WORKERS
CHUNK = 512
N_CHUNKS = B_PER_W // CHUNK


@functools.partial(
    pl.kernel,
    mesh=plsc.VectorSubcoreMesh(core_axis_name="c", subcore_axis_name="s"),
    out_type=jax.ShapeDtypeStruct((B_TOTAL, DIM), jnp.float32),
    scratch_types=[
        pltpu.VMEM((CHUNK,), jnp.int32),
        pltpu.VMEM((CHUNK, DIM), jnp.float32),
        pltpu.SemaphoreType.DMA,
    ],
    compiler_params=pltpu.CompilerParams(use_tc_tiling_on_sc=False),
)
def _gather_kernel(idx_hbm, table_hbm, out_hbm, idx_v, rows_v, sem):
    wid = lax.axis_index("s") * 2 + lax.axis_index("c")
    base = wid * B_PER_W

    def body(g, carry):
        off = base + g * CHUNK
        pltpu.sync_copy(idx_hbm.at[pl.ds(off, CHUNK)], idx_v)
        pltpu.async_copy(table_hbm.at[idx_v], rows_v, sem).wait()
        pltpu.sync_copy(rows_v, out_hbm.at[pl.ds(off, CHUNK)])
        return carry

    lax.fori_loop(0, N_CHUNKS, body, 0)


def kernel(input_, weight):
    idx = input_.reshape(-1).astype(jnp.int32)
    out = _gather_kernel(idx, weight)
    return out.reshape(BATCH, SEQ, DIM)


# speedup vs baseline: 1.0446x; 1.0446x over previous
import functools

import jax
import jax.numpy as jnp
from jax import lax
from jax.experimental import pallas as pl
from jax.experimental.pallas import tpu as pltpu
from jax.experimental.pallas import tpu_sc as plsc

BATCH = 4096
SEQ = 200
DIM = 64
B_TOTAL = BATCH * SEQ
NUM_WORKERS = 32
B_PER_W = B_TOTAL // NUM_WORKERS
CHUNK = 640
N_CHUNKS = B_PER_W // CHUNK
N_PAIRS = N_CHUNKS // 2


@functools.partial(
    pl.kernel,
    mesh=plsc.VectorSubcoreMesh(core_axis_name="c", subcore_axis_name="s"),
    out_type=jax.ShapeDtypeStruct((B_TOTAL, DIM), jnp.float32),
    scratch_types=[
        pltpu.VMEM((N_CHUNKS, CHUNK), jnp.int32),
        pltpu.VMEM((CHUNK, DIM), jnp.float32),
        pltpu.VMEM((CHUNK, DIM), jnp.float32),
        pltpu.SemaphoreType.DMA,
        pltpu.SemaphoreType.DMA,
        pltpu.SemaphoreType.DMA,
        pltpu.SemaphoreType.DMA,
    ],
    compiler_params=pltpu.CompilerParams(use_tc_tiling_on_sc=False),
)
def _gather_kernel(idx_hbm, table_hbm, out_hbm, idx_v, rows0, rows1,
                   gsem0, gsem1, wsem0, wsem1):
    wid = lax.axis_index("s") * 2 + lax.axis_index("c")
    base = wid * B_PER_W

    pltpu.sync_copy(idx_hbm.at[wid], idx_v)

    def gather(c, rows, sem):
        return pltpu.make_async_copy(table_hbm.at[idx_v.at[c]], rows, sem)

    def writeback(c, rows, sem):
        return pltpu.make_async_copy(
            rows, out_hbm.at[pl.ds(base + c * CHUNK, CHUNK)], sem)

    gather(0, rows0, gsem0).start()

    def body(k, carry):
        c0 = 2 * k
        @pl.when(k > 0)
        def _():
            writeback(c0 - 1, rows1, wsem1).wait()

        gather(c0 + 1, rows1, gsem1).start()
        gather(c0, rows0, gsem0).wait()
        writeback(c0, rows0, wsem0).start()

        @pl.when(k < N_PAIRS - 1)
        def _():
            writeback(c0, rows0, wsem0).wait()
            gather(c0 + 2, rows0, gsem0).start()

        gather(c0 + 1, rows1, gsem1).wait()
        writeback(c0 + 1, rows1, wsem1).start()
        return carry

    lax.fori_loop(0, N_PAIRS, body, 0)

    writeback(N_CHUNKS - 2, rows0, wsem0).wait()
    writeback(N_CHUNKS - 1, rows1, wsem1).wait()


def kernel(input_, weight):
    idx = input_.reshape(NUM_WORKERS, N_CHUNKS, CHUNK).astype(jnp.int32)
    out = _gather_kernel(idx, weight)
    return out.reshape(BATCH, SEQ, DIM)


# speedup vs baseline: 1.2748x; 1.2204x over previous
import functools

import jax
import jax.numpy as jnp
from jax import lax
from jax.experimental import pallas as pl
from jax.experimental.pallas import tpu as pltpu
from jax.experimental.pallas import tpu_sc as plsc

BATCH = 4096
SEQ = 200
DIM = 64
PDIM = 128
B_TOTAL = BATCH * SEQ
NUM_WORKERS = 32
B_PER_W = B_TOTAL // NUM_WORKERS
CHUNK = 320
N_CHUNKS = B_PER_W // CHUNK
N_PAIRS = N_CHUNKS // 2


@functools.partial(
    pl.kernel,
    mesh=plsc.VectorSubcoreMesh(core_axis_name="c", subcore_axis_name="s"),
    out_type=jax.ShapeDtypeStruct((B_TOTAL, PDIM), jnp.float32),
    scratch_types=[
        pltpu.VMEM((B_PER_W,), jnp.int32),
        pltpu.VMEM((CHUNK, PDIM), jnp.float32),
        pltpu.VMEM((CHUNK, PDIM), jnp.float32),
        pltpu.SemaphoreType.DMA,
        pltpu.SemaphoreType.DMA,
        pltpu.SemaphoreType.DMA,
        pltpu.SemaphoreType.DMA,
    ],
    compiler_params=pltpu.CompilerParams(use_tc_tiling_on_sc=True),
)
def _gather_kernel(idx_hbm, table_hbm, out_hbm, idx_v, rows0, rows1,
                   gsem0, gsem1, wsem0, wsem1):
    wid = lax.axis_index("s") * 2 + lax.axis_index("c")
    base = wid * B_PER_W

    pltpu.sync_copy(idx_hbm.at[pl.ds(base, B_PER_W)], idx_v)

    def gather(c, rows, sem):
        return pltpu.make_async_copy(
            table_hbm.at[idx_v.at[pl.ds(c * CHUNK, CHUNK)]], rows, sem)

    def writeback(c, rows, sem):
        return pltpu.make_async_copy(
            rows, out_hbm.at[pl.ds(base + c * CHUNK, CHUNK)], sem)

    gather(0, rows0, gsem0).start()

    def body(k, carry):
        c0 = 2 * k
        @pl.when(k > 0)
        def _():
            writeback(c0 - 1, rows1, wsem1).wait()

        gather(c0 + 1, rows1, gsem1).start()
        gather(c0, rows0, gsem0).wait()
        writeback(c0, rows0, wsem0).start()

        @pl.when(k < N_PAIRS - 1)
        def _():
            writeback(c0, rows0, wsem0).wait()
            gather(c0 + 2, rows0, gsem0).start()

        gather(c0 + 1, rows1, gsem1).wait()
        writeback(c0 + 1, rows1, wsem1).start()
        return carry

    lax.fori_loop(0, N_PAIRS, body, 0)

    writeback(N_CHUNKS - 2, rows0, wsem0).wait()
    writeback(N_CHUNKS - 1, rows1, wsem1).wait()


def kernel(input_, weight):
    wpad = jnp.pad(weight, ((0, 0), (0, PDIM - DIM)))
    idx = input_.reshape(B_TOTAL).astype(jnp.int32)
    out = _gather_kernel(idx, wpad)
    return out[:, :DIM].reshape(BATCH, SEQ, DIM)
